# baseline (speedup 1.0000x reference)
import jax
import jax.numpy as jnp
from jax.experimental import pallas as pl
from jax.experimental.pallas import tpu as pltpu

C_CHUNK = 2048
G_EXTRA = 2048
S_TOTAL = 2 * C_CHUNK + G_EXTRA
TAIL = S_TOTAL - C_CHUNK
ZB = 1024
N_TAIL = TAIL // ZB


def _body(k_ref, v_ref, fk_ref, K_ref, V_ref, FK_ref, zkv_ref, zfk_ref, sem):
    B = k_ref.shape[0]
    zkv_ref[...] = jnp.zeros(zkv_ref.shape, zkv_ref.dtype)
    zfk_ref[...] = jnp.zeros(zfk_ref.shape, zfk_ref.dtype)

    copies = []
    for b in range(B):
        copies.append(pltpu.make_async_copy(
            k_ref.at[b], K_ref.at[b, pl.ds(0, C_CHUNK)], sem))
        copies.append(pltpu.make_async_copy(
            v_ref.at[b], V_ref.at[b, pl.ds(0, C_CHUNK)], sem))
        copies.append(pltpu.make_async_copy(
            fk_ref.at[b], FK_ref.at[b, pl.ds(0, C_CHUNK)], sem))
        for t in range(N_TAIL):
            s0 = C_CHUNK + t * ZB
            copies.append(pltpu.make_async_copy(
                zkv_ref, K_ref.at[b, pl.ds(s0, ZB)], sem))
            copies.append(pltpu.make_async_copy(
                zkv_ref, V_ref.at[b, pl.ds(s0, ZB)], sem))
            copies.append(pltpu.make_async_copy(
                zfk_ref, FK_ref.at[b, pl.ds(s0, ZB)], sem))
    for c in copies:
        c.start()
    for c in copies:
        c.wait()


def kernel(k_c, v_c, fk_c):
    B, C, H, D = k_c.shape
    F = fk_c.shape[-1]

    K, V, FK = pl.pallas_call(
        _body,
        in_specs=[
            pl.BlockSpec(memory_space=pl.ANY),
            pl.BlockSpec(memory_space=pl.ANY),
            pl.BlockSpec(memory_space=pl.ANY),
        ],
        out_specs=[
            pl.BlockSpec(memory_space=pl.ANY),
            pl.BlockSpec(memory_space=pl.ANY),
            pl.BlockSpec(memory_space=pl.ANY),
        ],
        out_shape=[
            jax.ShapeDtypeStruct((B, S_TOTAL, H, D), k_c.dtype),
            jax.ShapeDtypeStruct((B, S_TOTAL, H, D), v_c.dtype),
            jax.ShapeDtypeStruct((B, S_TOTAL, H, F), fk_c.dtype),
        ],
        scratch_shapes=[
            pltpu.VMEM((ZB, H, D), k_c.dtype),
            pltpu.VMEM((ZB, H, F), fk_c.dtype),
            pltpu.SemaphoreType.DMA,
        ],
    )(k_c, v_c, fk_c)

    Hs = jnp.zeros((B, H, F, D), dtype=k_c.dtype)
    S = jnp.zeros((B, H, F), dtype=k_c.dtype)
    return (K, V, FK, Hs, S)


# speedup vs baseline: 1.0132x; 1.0132x over previous
import jax
import jax.numpy as jnp
from jax.experimental import pallas as pl
from jax.experimental.pallas import tpu as pltpu

C_CHUNK = 2048
G_EXTRA = 2048
S_TOTAL = 2 * C_CHUNK + G_EXTRA
TAIL = S_TOTAL - C_CHUNK
PIECE = 512
N_COPYP = C_CHUNK // PIECE
N_TAILP = TAIL // PIECE
NSEM = 16


def _body(k_ref, v_ref, fk_ref, K_ref, V_ref, FK_ref, zkv_ref, zfk_ref, sems):
    B = k_ref.shape[0]
    zkv_ref[...] = jnp.zeros(zkv_ref.shape, zkv_ref.dtype)
    zfk_ref[...] = jnp.zeros(zfk_ref.shape, zfk_ref.dtype)

    copies = []

    def add(src, dst):
        copies.append(pltpu.make_async_copy(src, dst, sems.at[len(copies) % NSEM]))

    for b in range(B):
        for t in range(N_COPYP):
            s0 = t * PIECE
            add(k_ref.at[b, pl.ds(s0, PIECE)], K_ref.at[b, pl.ds(s0, PIECE)])
            add(v_ref.at[b, pl.ds(s0, PIECE)], V_ref.at[b, pl.ds(s0, PIECE)])
            add(fk_ref.at[b, pl.ds(s0, PIECE)], FK_ref.at[b, pl.ds(s0, PIECE)])
        for t in range(N_TAILP):
            s0 = C_CHUNK + t * PIECE
            add(zkv_ref, K_ref.at[b, pl.ds(s0, PIECE)])
            add(zkv_ref, V_ref.at[b, pl.ds(s0, PIECE)])
            add(zfk_ref, FK_ref.at[b, pl.ds(s0, PIECE)])
    for c in copies:
        c.start()
    for c in copies:
        c.wait()


def kernel(k_c, v_c, fk_c):
    B, C, H, D = k_c.shape
    F = fk_c.shape[-1]

    K, V, FK = pl.pallas_call(
        _body,
        in_specs=[
            pl.BlockSpec(memory_space=pl.ANY),
            pl.BlockSpec(memory_space=pl.ANY),
            pl.BlockSpec(memory_space=pl.ANY),
        ],
        out_specs=[
            pl.BlockSpec(memory_space=pl.ANY),
            pl.BlockSpec(memory_space=pl.ANY),
            pl.BlockSpec(memory_space=pl.ANY),
        ],
        out_shape=[
            jax.ShapeDtypeStruct((B, S_TOTAL, H, D), k_c.dtype),
            jax.ShapeDtypeStruct((B, S_TOTAL, H, D), v_c.dtype),
            jax.ShapeDtypeStruct((B, S_TOTAL, H, F), fk_c.dtype),
        ],
        scratch_shapes=[
            pltpu.VMEM((PIECE, H, D), k_c.dtype),
            pltpu.VMEM((PIECE, H, F), fk_c.dtype),
            pltpu.SemaphoreType.DMA((NSEM,)),
        ],
    )(k_c, v_c, fk_c)

    Hs = jnp.zeros((B, H, F, D), dtype=k_c.dtype)
    S = jnp.zeros((B, H, F), dtype=k_c.dtype)
    return (K, V, FK, Hs, S)


# speedup vs baseline: 14.2979x; 14.1119x over previous
import jax
import jax.numpy as jnp
from jax.experimental import pallas as pl

C_CHUNK = 2048
G_EXTRA = 2048
S_TOTAL = 2 * C_CHUNK + G_EXTRA

BLOCK_S = 512
N_BLOCKS = S_TOTAL // BLOCK_S
N_COPY = C_CHUNK // BLOCK_S


def _body(k_ref, v_ref, fk_ref, K_ref, V_ref, FK_ref):
    j = pl.program_id(1)

    @pl.when(j < N_COPY)
    def _copy():
        K_ref[...] = k_ref[...]
        V_ref[...] = v_ref[...]
        FK_ref[...] = fk_ref[...]

    @pl.when(j >= N_COPY)
    def _zero():
        K_ref[...] = jnp.zeros(K_ref.shape, K_ref.dtype)
        V_ref[...] = jnp.zeros(V_ref.shape, V_ref.dtype)
        FK_ref[...] = jnp.zeros(FK_ref.shape, FK_ref.dtype)


def kernel(k_c, v_c, fk_c):
    B, C, H, D = k_c.shape
    F = fk_c.shape[-1]

    def in_map(b, j):
        return (b, jnp.minimum(j, N_COPY - 1), 0, 0)

    def out_map(b, j):
        return (b, j, 0, 0)

    K, V, FK = pl.pallas_call(
        _body,
        grid=(B, N_BLOCKS),
        in_specs=[
            pl.BlockSpec((1, BLOCK_S, H, D), in_map),
            pl.BlockSpec((1, BLOCK_S, H, D), in_map),
            pl.BlockSpec((1, BLOCK_S, H, F), in_map),
        ],
        out_specs=[
            pl.BlockSpec((1, BLOCK_S, H, D), out_map),
            pl.BlockSpec((1, BLOCK_S, H, D), out_map),
            pl.BlockSpec((1, BLOCK_S, H, F), out_map),
        ],
        out_shape=[
            jax.ShapeDtypeStruct((B, S_TOTAL, H, D), k_c.dtype),
            jax.ShapeDtypeStruct((B, S_TOTAL, H, D), v_c.dtype),
            jax.ShapeDtypeStruct((B, S_TOTAL, H, F), fk_c.dtype),
        ],
    )(k_c, v_c, fk_c)

    Hs = jnp.zeros((B, H, F, D), dtype=k_c.dtype)
    S = jnp.zeros((B, H, F), dtype=k_c.dtype)
    return (K, V, FK, Hs, S)
